# baseline (speedup 1.0000x reference)
import jax
import jax.numpy as jnp
from jax import lax
from jax.experimental import pallas as pl
from jax.experimental.pallas import tpu as pltpu
from jax.experimental.pallas import tpu_sc as plsc

VOCAB = 50257
H = 1280
S = 1024
B = 32

NUM_CORES = 2
NUM_SUBCORES = 16
NW = NUM_CORES * NUM_SUBCORES
SCHUNK = S // NW
LANES = 16
GROUPS = H // LANES

UNIT = 16
UPC = SCHUNK // UNIT
NU = B * UPC
NBUF = 4
DIST = 3
T = NU // NBUF


def _body(ids_hbm, wte_hbm, wpe_hbm, out_hbm,
          idx_all, wpe_v, bufs, gsems, wsems):
  wid = lax.axis_index("s") * NUM_CORES + lax.axis_index("c")
  s0 = wid * SCHUNK

  def g_start(u, k):
    pltpu.async_copy(wte_hbm.at[idx_all.at[pl.ds(u * UNIT, UNIT)]],
                     bufs[k], gsems[k])

  def g_wait(u, k):
    pltpu.make_async_copy(wte_hbm.at[idx_all.at[pl.ds(u * UNIT, UNIT)]],
                          bufs[k], gsems[k]).wait()

  def out_slice(u, k):
    return out_hbm.at[pl.ds((u // UPC) * S + s0 + (k % UPC) * UNIT, UNIT)]

  def w_start(u, k):
    pltpu.async_copy(bufs[k], out_slice(u, k), wsems[k])

  def w_wait(u, k):
    pltpu.make_async_copy(bufs[k], out_slice(u, k), wsems[k]).wait()

  def add_unit(k):
    h = (k % UPC) * UNIT
    buf = bufs[k]

    @pl.loop(0, UNIT)
    def _(r):
      for j in range(GROUPS):
        plsc.addupdate(buf.at[r, pl.ds(j * LANES, LANES)],
                       wpe_v[h + r, pl.ds(j * LANES, LANES)])

  pltpu.sync_copy(ids_hbm.at[pl.ds(wid * B * SCHUNK, B * SCHUNK)], idx_all)
  pltpu.sync_copy(wpe_hbm.at[pl.ds(s0, SCHUNK)], wpe_v)

  g_start(0, 0)
  g_start(1, 1)
  g_start(2, 2)

  @pl.loop(0, T)
  def _(t):
    u0 = NBUF * t
    for k in range(NBUF):
      u = u0 + k
      kn = (k + DIST) % NBUF
      g_wait(u, k)
      add_unit(k)
      w_start(u, k)
      if k == 0:
        @pl.when(t > 0)
        def _():
          w_wait(u - 1, kn)
        g_start(u + DIST, kn)
      else:
        @pl.when(t < T - 1)
        def _():
          w_wait(u - 1, kn)
          g_start(u + DIST, kn)

  w_wait(NU - 4, 0)
  w_wait(NU - 3, 1)
  w_wait(NU - 2, 2)
  w_wait(NU - 1, 3)


@jax.jit
def kernel(input_ids, wte, wpe):
  ids = (input_ids.astype(jnp.int32)
         .reshape(B, NW, SCHUNK).swapaxes(0, 1).reshape(-1))
  mesh = plsc.VectorSubcoreMesh(core_axis_name="c", subcore_axis_name="s")
  run = pl.kernel(
      _body,
      out_type=jax.ShapeDtypeStruct((B * S, H), jnp.float32),
      mesh=mesh,
      scratch_types=[
          pltpu.VMEM((B * SCHUNK,), jnp.int32),
          pltpu.VMEM((SCHUNK, H), jnp.float32),
          [pltpu.VMEM((UNIT, H), jnp.float32) for _ in range(NBUF)],
          [pltpu.SemaphoreType.DMA for _ in range(NBUF)],
          [pltpu.SemaphoreType.DMA for _ in range(NBUF)],
      ],
  )
  out = run(ids, wte, wpe)
  return out.reshape(B, S, H)


# speedup vs baseline: 1.0519x; 1.0519x over previous
import jax
import jax.numpy as jnp
from jax import lax
from jax.experimental import pallas as pl
from jax.experimental.pallas import tpu as pltpu
from jax.experimental.pallas import tpu_sc as plsc

VOCAB = 50257
H = 1280
S = 1024
B = 32

NUM_CORES = 2
NUM_SUBCORES = 16
NW = NUM_CORES * NUM_SUBCORES
SCHUNK = S // NW
HALF = SCHUNK // 2
LANES = 16
GROUPS = H // LANES


def _body(ids_hbm, wte_hbm, wpe_hbm, out_hbm,
          idx_all, wpe_v, rows0, rows1, gsem0, gsem1, wsem0, wsem1):
  wid = lax.axis_index("s") * NUM_CORES + lax.axis_index("c")
  s0 = wid * SCHUNK

  def g_start(b, buf, sem):
    pltpu.async_copy(wte_hbm.at[idx_all.at[pl.ds(b * SCHUNK, SCHUNK)]],
                     buf, sem)

  def g_wait(b, buf, sem):
    pltpu.make_async_copy(wte_hbm.at[idx_all.at[pl.ds(b * SCHUNK, SCHUNK)]],
                          buf, sem).wait()

  def w_start_half(b, buf, sem, h):
    pltpu.async_copy(buf.at[pl.ds(h * HALF, HALF)],
                     out_hbm.at[pl.ds(b * S + s0 + h * HALF, HALF)], sem)

  def w_wait_all(b, buf, sem):
    for h in range(2):
      pltpu.make_async_copy(
          buf.at[pl.ds(h * HALF, HALF)],
          out_hbm.at[pl.ds(b * S + s0 + h * HALF, HALF)], sem).wait()

  def add_half(buf, h):
    @pl.loop(h * HALF, (h + 1) * HALF)
    def _(r):
      for j in range(GROUPS):
        plsc.addupdate(buf.at[r, pl.ds(j * LANES, LANES)],
                       wpe_v[r, pl.ds(j * LANES, LANES)])

  def process(b, buf, sem):
    add_half(buf, 0)
    w_start_half(b, buf, sem, 0)
    add_half(buf, 1)
    w_start_half(b, buf, sem, 1)

  pltpu.sync_copy(ids_hbm.at[pl.ds(wid * B * SCHUNK, B * SCHUNK)], idx_all)
  pltpu.sync_copy(wpe_hbm.at[pl.ds(s0, SCHUNK)], wpe_v)

  g_start(0, rows0, gsem0)

  @pl.loop(0, B // 2)
  def _(t):
    b0 = 2 * t
    b1 = 2 * t + 1

    @pl.when(t > 0)
    def _():
      w_wait_all(b1, rows1, wsem1)

    g_start(b1, rows1, gsem1)
    g_wait(b0, rows0, gsem0)
    process(b0, rows0, wsem0)

    g_wait(b1, rows1, gsem1)

    @pl.when(t < B // 2 - 1)
    def _():
      w_wait_all(b0, rows0, wsem0)
      g_start(b0 + 2, rows0, gsem0)

    process(b1, rows1, wsem1)

  w_wait_all(B - 2, rows0, wsem0)
  w_wait_all(B - 1, rows1, wsem1)


@jax.jit
def kernel(input_ids, wte, wpe):
  ids = (input_ids.astype(jnp.int32)
         .reshape(B, NW, SCHUNK).swapaxes(0, 1).reshape(-1))
  mesh = plsc.VectorSubcoreMesh(core_axis_name="c", subcore_axis_name="s")
  run = pl.kernel(
      _body,
      out_type=jax.ShapeDtypeStruct((B * S, H), jnp.float32),
      mesh=mesh,
      scratch_types=[
          pltpu.VMEM((B * SCHUNK,), jnp.int32),
          pltpu.VMEM((SCHUNK, H), jnp.float32),
          pltpu.VMEM((SCHUNK, H), jnp.float32),
          pltpu.VMEM((SCHUNK, H), jnp.float32),
          pltpu.SemaphoreType.DMA,
          pltpu.SemaphoreType.DMA,
          pltpu.SemaphoreType.DMA,
          pltpu.SemaphoreType.DMA,
      ],
  )
  out = run(ids, wte, wpe)
  return out.reshape(B, S, H)


# speedup vs baseline: 1.1360x; 1.0800x over previous
import jax
import jax.numpy as jnp
from jax import lax
from jax.experimental import pallas as pl
from jax.experimental.pallas import tpu as pltpu
from jax.experimental.pallas import tpu_sc as plsc

VOCAB = 50257
H = 1280
S = 1024
B = 32

NUM_CORES = 2
NUM_SUBCORES = 16
NW = NUM_CORES * NUM_SUBCORES
SCHUNK = S // NW
LANES = 16
GROUPS = H // LANES


def _body(ids_hbm, wte_hbm, wpe_hbm, out_hbm,
          idx_all, wpe_v, rows0, rows1, gsem0, gsem1, wsem0, wsem1):
  wid = lax.axis_index("s") * NUM_CORES + lax.axis_index("c")
  s0 = wid * SCHUNK
  bufs = (rows0, rows1)
  gsems = (gsem0, gsem1)
  wsems = (wsem0, wsem1)

  def g_start(b, p):
    pltpu.async_copy(wte_hbm.at[idx_all.at[pl.ds(b * SCHUNK, SCHUNK)]],
                     bufs[p], gsems[p])

  def g_wait(b, p):
    pltpu.make_async_copy(wte_hbm.at[idx_all.at[pl.ds(b * SCHUNK, SCHUNK)]],
                          bufs[p], gsems[p]).wait()

  def w_start(b, p):
    pltpu.async_copy(bufs[p], out_hbm.at[pl.ds(b * S + s0, SCHUNK)],
                     wsems[p])

  def w_wait(b, p):
    pltpu.make_async_copy(
        bufs[p], out_hbm.at[pl.ds(b * S + s0, SCHUNK)], wsems[p]).wait()

  def add_chunk(p):
    buf = bufs[p]

    @pl.loop(0, SCHUNK)
    def _(r):
      for j in range(GROUPS):
        plsc.addupdate(buf.at[r, pl.ds(j * LANES, LANES)],
                       wpe_v[r, pl.ds(j * LANES, LANES)])

  pltpu.sync_copy(ids_hbm.at[pl.ds(wid * B * SCHUNK, B * SCHUNK)], idx_all)
  pltpu.sync_copy(wpe_hbm.at[pl.ds(s0, SCHUNK)], wpe_v)

  g_start(0, 0)

  @pl.loop(0, B // 2)
  def _(t):
    for j in range(2):
      b = 2 * t + j
      p = j
      g_wait(b, p)
      if j == 0:
        @pl.when(t > 0)
        def _():
          w_wait(b - 1, 1 - p)
        g_start(b + 1, 1 - p)
      else:
        @pl.when(t < B // 2 - 1)
        def _():
          w_wait(b - 1, 1 - p)
          g_start(b + 1, 1 - p)
      add_chunk(p)
      w_start(b, p)

  w_wait(B - 2, 0)
  w_wait(B - 1, 1)


@jax.jit
def kernel(input_ids, wte, wpe):
  ids = (input_ids.astype(jnp.int32)
         .reshape(B, NW, SCHUNK).swapaxes(0, 1).reshape(-1))
  mesh = plsc.VectorSubcoreMesh(core_axis_name="c", subcore_axis_name="s")
  run = pl.kernel(
      _body,
      out_type=jax.ShapeDtypeStruct((B * S, H), jnp.float32),
      mesh=mesh,
      scratch_types=[
          pltpu.VMEM((B * SCHUNK,), jnp.int32),
          pltpu.VMEM((SCHUNK, H), jnp.float32),
          pltpu.VMEM((SCHUNK, H), jnp.float32),
          pltpu.VMEM((SCHUNK, H), jnp.float32),
          pltpu.SemaphoreType.DMA,
          pltpu.SemaphoreType.DMA,
          pltpu.SemaphoreType.DMA,
          pltpu.SemaphoreType.DMA,
      ],
  )
  out = run(ids, wte, wpe)
  return out.reshape(B, S, H)
